# baseline (speedup 1.0000x reference)
import functools

import jax
import jax.numpy as jnp
from jax import lax
from jax.experimental import pallas as pl
from jax.experimental.pallas import tpu as pltpu
from jax.experimental.pallas import tpu_sc as plsc

_ILV = plsc.PackFormat.INTERLEAVED

CROP = 7
NPTS = CROP * CROP
CH = 256
LVL_DIM = (128, 64, 32, 16)
LVL_OFF = (0, 128 * 128, 128 * 128 + 64 * 64, 128 * 128 + 64 * 64 + 32 * 32)
LVL_SCALE = (0.25, 0.125, 0.0625, 0.03125)
N_ROI = 1000
N_WORKERS = 32
ROI_PER_W = 32
IDX_A = 112
IDX_B = 96
G_ROWS = IDX_A + IDX_B


def _vgather(x, idx):
    dnums = lax.GatherDimensionNumbers(
        offset_dims=(), collapsed_slice_dims=(0,), start_index_map=(0,))
    return lax.gather(
        x, idx[:, None], dnums, (1,),
        mode=lax.GatherScatterMode.PROMISE_IN_BOUNDS)


def _splat(v, dtype=jnp.float32):
    return jnp.full((16,), v, dtype=dtype)


def _axis_coords(lo, hi, scale, dim_i):
    los = lo * scale
    his = hi * scale
    bw = (his - los) * (1.0 / CROP)
    pos = lax.broadcasted_iota(jnp.int32, (16,), 0).astype(jnp.float32)
    xs = los + (pos + 0.5) * bw - 0.5
    t = xs.astype(jnp.int32)
    tf = t.astype(jnp.float32)
    xf = t - jnp.where(xs < tf, 1, 0)
    lx = xs - xf.astype(jnp.float32)
    hx = 1.0 - lx
    zero = jnp.zeros((16,), jnp.int32)
    x0i = jnp.maximum(jnp.minimum(xf, dim_i - 1), zero)
    x1i = jnp.maximum(jnp.minimum(xf + 1, dim_i - 1), zero)
    cs = jnp.maximum(jnp.minimum(xf, dim_i - 2), zero)
    fz = jnp.zeros((16,), jnp.float32)
    wa = jnp.where(x0i == cs, hx, fz) + jnp.where(x1i == cs, lx, fz)
    wb = jnp.where(x0i == cs + 1, hx, fz) + jnp.where(x1i == cs + 1, lx, fz)
    return cs, wa, wb


def _sc_body(table_hbm, props_hbm, out_hbm, props_v, idx_a0, idx_b0, idx_a1,
             idx_b1, gbuf, obuf, wbuf, sem0, sem1):
    nc = 2
    wid = lax.axis_index("s") * nc + lax.axis_index("c")
    base = wid * ROI_PER_W
    pltpu.sync_copy(props_hbm.at[pl.ds(base * 8, ROI_PER_W * 8 + 16)],
                    props_v)
    slot_refs = ((idx_a0, idx_b0, 0, sem0), (idx_a1, idx_b1, 1, sem1))

    def prefetch(r, slot):
        idx_a, idx_b, goff, sem = slot_refs[slot]
        rloc = r - base
        row = props_v[pl.ds(rloc * 8, 16)]
        x0 = _vgather(row, _splat(1, jnp.int32))
        y0 = _vgather(row, _splat(2, jnp.int32))
        x1 = _vgather(row, _splat(3, jnp.int32))
        y1 = _vgather(row, _splat(4, jnp.int32))

        area = (x1 - x0) * (y1 - y0)
        lvl = (jnp.where(area > 48.0 * 48.0, 1, 0)
               + jnp.where(area > 96.0 * 96.0, 1, 0)
               + jnp.where(area > 192.0 * 192.0, 1, 0))
        dim_i = jnp.where(
            lvl == 0, LVL_DIM[0],
            jnp.where(lvl == 1, LVL_DIM[1],
                      jnp.where(lvl == 2, LVL_DIM[2], LVL_DIM[3])))
        off = jnp.where(
            lvl == 0, LVL_OFF[0],
            jnp.where(lvl == 1, LVL_OFF[1],
                      jnp.where(lvl == 2, LVL_OFF[2], LVL_OFF[3])))
        scale = jnp.where(
            lvl == 0, LVL_SCALE[0],
            jnp.where(lvl == 1, LVL_SCALE[1],
                      jnp.where(lvl == 2, LVL_SCALE[2], LVL_SCALE[3])))

        cs, wa, wb = _axis_coords(x0, x1, scale, dim_i)
        rs, va, vb = _axis_coords(y0, y1, scale, dim_i)
        wbuf[pl.ds(slot * 64 + 0, 16)] = va
        wbuf[pl.ds(slot * 64 + 16, 16)] = vb
        wbuf[pl.ds(slot * 64 + 32, 16)] = wa
        wbuf[pl.ds(slot * 64 + 48, 16)] = wb

        lane = lax.broadcasted_iota(jnp.int32, (16,), 0)
        nsel = lane - (lane // 4) * 4
        dy_t = nsel // 2
        dx_t = nsel - dy_t * 2
        for c in range(13):
            pts = jnp.minimum(lane // 4 + 4 * c, NPTS - 1)
            iv = pts // CROP
            jv = pts - iv * CROP
            rg = _vgather(rs, iv)
            cg = _vgather(cs, jv)
            idx = off + (rg + dy_t) * dim_i + cg + dx_t
            if c < 7:
                idx_a[pl.ds(16 * c, 16)] = idx
            else:
                idx_b[pl.ds(16 * (c - 7), 16)] = idx

        pltpu.async_copy(table_hbm.at[idx_a],
                         gbuf.at[pl.ds(goff * G_ROWS, IDX_A)], sem)
        pltpu.async_copy(table_hbm.at[idx_b],
                         gbuf.at[pl.ds(goff * G_ROWS + IDX_A, IDX_B)], sem)

    def wait_slot(slot):
        idx_a, idx_b, goff, sem = slot_refs[slot]
        pltpu.make_async_copy(table_hbm.at[idx_a],
                              gbuf.at[pl.ds(goff * G_ROWS, IDX_A)],
                              sem).wait()
        pltpu.make_async_copy(table_hbm.at[idx_b],
                              gbuf.at[pl.ds(goff * G_ROWS + IDX_A, IDX_B)],
                              sem).wait()

    def compute(r, slot):
        goff = slot * G_ROWS
        woff = slot * 64

        def pt_body(k, _):
            i = k // CROP
            j = k - i * CROP
            ivec = _splat(i, jnp.int32)
            jvec = _splat(j, jnp.int32)
            vai = _vgather(wbuf[pl.ds(woff + 0, 16)], ivec)
            vbi = _vgather(wbuf[pl.ds(woff + 16, 16)], ivec)
            waj = _vgather(wbuf[pl.ds(woff + 32, 16)], jvec)
            wbj = _vgather(wbuf[pl.ds(woff + 48, 16)], jvec)
            w00 = vai * waj
            w01 = vai * wbj
            w10 = vbi * waj
            w11 = vbi * wbj
            r0 = goff + 4 * k
            kv = _splat(k, jnp.int32)
            cio = lax.broadcasted_iota(jnp.int32, (16,), 0)
            for c in range(8):
                sl = pl.ds(16 * c, 16)
                bc = lambda v: plsc.bitcast(v, jnp.bfloat16)
                p00a, p00b = plsc.unpack(bc(gbuf[r0, sl]), format=_ILV)
                p01a, p01b = plsc.unpack(bc(gbuf[r0 + 1, sl]), format=_ILV)
                p10a, p10b = plsc.unpack(bc(gbuf[r0 + 2, sl]), format=_ILV)
                p11a, p11b = plsc.unpack(bc(gbuf[r0 + 3, sl]), format=_ILV)
                acc_a = (p00a * w00 + p01a * w01 + p10a * w10 + p11a * w11)
                acc_b = (p00b * w00 + p01b * w01 + p10b * w10 + p11b * w11)
                plsc.store_scatter(obuf, [cio + 32 * c, kv], acc_a)
                plsc.store_scatter(obuf, [cio + 32 * c + 16, kv], acc_b)
            return 0

        lax.fori_loop(0, NPTS, pt_body, 0, unroll=False)
        pltpu.sync_copy(obuf, out_hbm.at[r])

    end = jnp.minimum(base + ROI_PER_W, N_ROI)
    prefetch(base, 0)

    def pair_body(t, _):
        r0 = base + 2 * t
        r1 = r0 + 1
        wait_slot(0)
        prefetch(r1, 1)
        compute(r0, 0)

        @pl.when(r0 + 2 < end)
        def _():
            prefetch(r0 + 2, 0)

        wait_slot(1)
        compute(r1, 1)
        return 0

    lax.fori_loop(0, (end - base) // 2, pair_body, 0, unroll=False)


@jax.jit
def kernel(f2, f3, f4, f5, proposals):
    levels = [f2, f3, f4, f5]
    table = jnp.concatenate(
        [jnp.transpose(f[0].reshape(CH, -1)) for f in levels], axis=0)
    table = (table.reshape(-1, 8, 2, 16).swapaxes(2, 3)
             .reshape(-1, CH).astype(jnp.bfloat16))
    table = lax.bitcast_convert_type(
        table.reshape(-1, CH // 2, 2), jnp.int32)
    props = jnp.concatenate(
        [proposals.astype(jnp.float32),
         jnp.zeros((N_WORKERS * ROI_PER_W - N_ROI, 7), jnp.float32)], axis=0)
    props_flat = jnp.concatenate(
        [jnp.pad(props, ((0, 0), (0, 1))).reshape(-1),
         jnp.zeros((16,), jnp.float32)])

    run = functools.partial(
        pl.kernel,
        mesh=plsc.VectorSubcoreMesh(core_axis_name="c", subcore_axis_name="s"),
        compiler_params=pltpu.CompilerParams(needs_layout_passes=False),
        out_type=jax.ShapeDtypeStruct((N_ROI, CH, NPTS), jnp.float32),
        scratch_types=[
            pltpu.VMEM((ROI_PER_W * 8 + 16,), jnp.float32),
            pltpu.VMEM((IDX_A,), jnp.int32),
            pltpu.VMEM((IDX_B,), jnp.int32),
            pltpu.VMEM((IDX_A,), jnp.int32),
            pltpu.VMEM((IDX_B,), jnp.int32),
            pltpu.VMEM((2 * G_ROWS, CH // 2), jnp.int32),
            pltpu.VMEM((CH, NPTS), jnp.float32),
            pltpu.VMEM((128,), jnp.float32),
            pltpu.SemaphoreType.DMA,
            pltpu.SemaphoreType.DMA,
        ],
    )(_sc_body)
    out = run(table, props_flat)
    return out.reshape(N_ROI, CH, CROP, CROP)


# speedup vs baseline: 1.4699x; 1.4699x over previous
import functools

import jax
import jax.numpy as jnp
from jax import lax
from jax.experimental import pallas as pl
from jax.experimental.pallas import tpu as pltpu
from jax.experimental.pallas import tpu_sc as plsc

_ILV = plsc.PackFormat.INTERLEAVED

CROP = 7
NPTS = CROP * CROP
CH = 256
LVL_DIM = (128, 64, 32, 16)
LVL_OFF = (0, 128 * 128, 128 * 128 + 64 * 64, 128 * 128 + 64 * 64 + 32 * 32)
LVL_SCALE = (0.25, 0.125, 0.0625, 0.03125)
N_ROI = 1000
N_WORKERS = 32
ROI_PER_W = 32
IDX_A = 112
IDX_B = 96
G_ROWS = IDX_A + IDX_B


def _vgather(x, idx):
    dnums = lax.GatherDimensionNumbers(
        offset_dims=(), collapsed_slice_dims=(0,), start_index_map=(0,))
    return lax.gather(
        x, idx[:, None], dnums, (1,),
        mode=lax.GatherScatterMode.PROMISE_IN_BOUNDS)


def _splat(v, dtype=jnp.float32):
    return jnp.full((16,), v, dtype=dtype)


def _axis_coords(lo, hi, scale, dim_i):
    los = lo * scale
    his = hi * scale
    bw = (his - los) * (1.0 / CROP)
    pos = lax.broadcasted_iota(jnp.int32, (16,), 0).astype(jnp.float32)
    xs = los + (pos + 0.5) * bw - 0.5
    t = xs.astype(jnp.int32)
    tf = t.astype(jnp.float32)
    xf = t - jnp.where(xs < tf, 1, 0)
    lx = xs - xf.astype(jnp.float32)
    hx = 1.0 - lx
    zero = jnp.zeros((16,), jnp.int32)
    x0i = jnp.maximum(jnp.minimum(xf, dim_i - 1), zero)
    x1i = jnp.maximum(jnp.minimum(xf + 1, dim_i - 1), zero)
    cs = jnp.maximum(jnp.minimum(xf, dim_i - 2), zero)
    fz = jnp.zeros((16,), jnp.float32)
    wa = jnp.where(x0i == cs, hx, fz) + jnp.where(x1i == cs, lx, fz)
    wb = jnp.where(x0i == cs + 1, hx, fz) + jnp.where(x1i == cs + 1, lx, fz)
    return cs, wa, wb


def _sc_body(table_hbm, props_hbm, out_hbm, props_v, idx_a0, idx_b0, idx_a1,
             idx_b1, gbuf, obuf, wbuf, sem0, sem1):
    nc = 2
    wid = lax.axis_index("s") * nc + lax.axis_index("c")
    base = wid * ROI_PER_W
    pltpu.sync_copy(props_hbm.at[pl.ds(base * 8, ROI_PER_W * 8 + 16)],
                    props_v)
    slot_refs = ((idx_a0, idx_b0, 0, sem0), (idx_a1, idx_b1, 1, sem1))

    def prefetch(r, slot):
        idx_a, idx_b, goff, sem = slot_refs[slot]
        rloc = r - base
        row = props_v[pl.ds(rloc * 8, 16)]
        x0 = _vgather(row, _splat(1, jnp.int32))
        y0 = _vgather(row, _splat(2, jnp.int32))
        x1 = _vgather(row, _splat(3, jnp.int32))
        y1 = _vgather(row, _splat(4, jnp.int32))

        area = (x1 - x0) * (y1 - y0)
        lvl = (jnp.where(area > 48.0 * 48.0, 1, 0)
               + jnp.where(area > 96.0 * 96.0, 1, 0)
               + jnp.where(area > 192.0 * 192.0, 1, 0))
        dim_i = jnp.where(
            lvl == 0, LVL_DIM[0],
            jnp.where(lvl == 1, LVL_DIM[1],
                      jnp.where(lvl == 2, LVL_DIM[2], LVL_DIM[3])))
        off = jnp.where(
            lvl == 0, LVL_OFF[0],
            jnp.where(lvl == 1, LVL_OFF[1],
                      jnp.where(lvl == 2, LVL_OFF[2], LVL_OFF[3])))
        scale = jnp.where(
            lvl == 0, LVL_SCALE[0],
            jnp.where(lvl == 1, LVL_SCALE[1],
                      jnp.where(lvl == 2, LVL_SCALE[2], LVL_SCALE[3])))

        cs, wa, wb = _axis_coords(x0, x1, scale, dim_i)
        rs, va, vb = _axis_coords(y0, y1, scale, dim_i)
        wbuf[pl.ds(slot * 64 + 0, 16)] = va
        wbuf[pl.ds(slot * 64 + 16, 16)] = vb
        wbuf[pl.ds(slot * 64 + 32, 16)] = wa
        wbuf[pl.ds(slot * 64 + 48, 16)] = wb

        lane = lax.broadcasted_iota(jnp.int32, (16,), 0)
        nsel = lane - (lane // 4) * 4
        dy_t = nsel // 2
        dx_t = nsel - dy_t * 2
        for c in range(13):
            pts = jnp.minimum(lane // 4 + 4 * c, NPTS - 1)
            iv = pts // CROP
            jv = pts - iv * CROP
            rg = _vgather(rs, iv)
            cg = _vgather(cs, jv)
            idx = off + (rg + dy_t) * dim_i + cg + dx_t
            if c < 7:
                idx_a[pl.ds(16 * c, 16)] = idx
            else:
                idx_b[pl.ds(16 * (c - 7), 16)] = idx

        pltpu.async_copy(table_hbm.at[idx_a],
                         gbuf.at[pl.ds(goff * G_ROWS, IDX_A)], sem)
        pltpu.async_copy(table_hbm.at[idx_b],
                         gbuf.at[pl.ds(goff * G_ROWS + IDX_A, IDX_B)], sem)

    def wait_slot(slot):
        idx_a, idx_b, goff, sem = slot_refs[slot]
        pltpu.make_async_copy(table_hbm.at[idx_a],
                              gbuf.at[pl.ds(goff * G_ROWS, IDX_A)],
                              sem).wait()
        pltpu.make_async_copy(table_hbm.at[idx_b],
                              gbuf.at[pl.ds(goff * G_ROWS + IDX_A, IDX_B)],
                              sem).wait()

    def compute(r, slot):
        goff = slot * G_ROWS
        woff = slot * 64

        def pt_body(k, _):
            i = k // CROP
            j = k - i * CROP
            ivec = _splat(i, jnp.int32)
            jvec = _splat(j, jnp.int32)
            vai = _vgather(wbuf[pl.ds(woff + 0, 16)], ivec)
            vbi = _vgather(wbuf[pl.ds(woff + 16, 16)], ivec)
            waj = _vgather(wbuf[pl.ds(woff + 32, 16)], jvec)
            wbj = _vgather(wbuf[pl.ds(woff + 48, 16)], jvec)
            w00 = vai * waj
            w01 = vai * wbj
            w10 = vbi * waj
            w11 = vbi * wbj
            r0 = goff + 4 * k
            soff = lax.broadcasted_iota(jnp.int32, (16,), 0) * NPTS + k
            for c in range(8):
                sl = pl.ds(16 * c, 16)
                bc = lambda v: plsc.bitcast(v, jnp.bfloat16)
                p00a, p00b = plsc.unpack(bc(gbuf[r0, sl]), format=_ILV)
                p01a, p01b = plsc.unpack(bc(gbuf[r0 + 1, sl]), format=_ILV)
                p10a, p10b = plsc.unpack(bc(gbuf[r0 + 2, sl]), format=_ILV)
                p11a, p11b = plsc.unpack(bc(gbuf[r0 + 3, sl]), format=_ILV)
                acc_a = (p00a * w00 + p01a * w01 + p10a * w10 + p11a * w11)
                acc_b = (p00b * w00 + p01b * w01 + p10b * w10 + p11b * w11)
                plsc.store_scatter(obuf, [soff + 32 * NPTS * c], acc_a)
                plsc.store_scatter(obuf, [soff + 32 * NPTS * c + 16 * NPTS],
                                   acc_b)
            return 0

        lax.fori_loop(0, NPTS, pt_body, 0, unroll=False)
        pltpu.sync_copy(obuf, out_hbm.at[r])

    end = jnp.minimum(base + ROI_PER_W, N_ROI)
    prefetch(base, 0)

    def pair_body(t, _):
        r0 = base + 2 * t
        r1 = r0 + 1
        wait_slot(0)
        prefetch(r1, 1)
        compute(r0, 0)

        @pl.when(r0 + 2 < end)
        def _():
            prefetch(r0 + 2, 0)

        wait_slot(1)
        compute(r1, 1)
        return 0

    lax.fori_loop(0, (end - base) // 2, pair_body, 0, unroll=False)


@jax.jit
def kernel(f2, f3, f4, f5, proposals):
    levels = [f2, f3, f4, f5]
    table = jnp.concatenate(
        [jnp.transpose(f[0].reshape(CH, -1)) for f in levels], axis=0)
    table = (table.reshape(-1, 8, 2, 16).swapaxes(2, 3)
             .reshape(-1, CH).astype(jnp.bfloat16))
    table = lax.bitcast_convert_type(
        table.reshape(-1, CH // 2, 2), jnp.int32)
    props = jnp.concatenate(
        [proposals.astype(jnp.float32),
         jnp.zeros((N_WORKERS * ROI_PER_W - N_ROI, 7), jnp.float32)], axis=0)
    props_flat = jnp.concatenate(
        [jnp.pad(props, ((0, 0), (0, 1))).reshape(-1),
         jnp.zeros((16,), jnp.float32)])

    run = functools.partial(
        pl.kernel,
        mesh=plsc.VectorSubcoreMesh(core_axis_name="c", subcore_axis_name="s"),
        compiler_params=pltpu.CompilerParams(needs_layout_passes=False),
        out_type=jax.ShapeDtypeStruct((N_ROI, CH * NPTS), jnp.float32),
        scratch_types=[
            pltpu.VMEM((ROI_PER_W * 8 + 16,), jnp.float32),
            pltpu.VMEM((IDX_A,), jnp.int32),
            pltpu.VMEM((IDX_B,), jnp.int32),
            pltpu.VMEM((IDX_A,), jnp.int32),
            pltpu.VMEM((IDX_B,), jnp.int32),
            pltpu.VMEM((2 * G_ROWS, CH // 2), jnp.int32),
            pltpu.VMEM((CH * NPTS,), jnp.float32),
            pltpu.VMEM((128,), jnp.float32),
            pltpu.SemaphoreType.DMA,
            pltpu.SemaphoreType.DMA,
        ],
    )(_sc_body)
    out = run(table, props_flat)
    return out.reshape(N_ROI, CH, CROP, CROP)
